# baseline (speedup 1.0000x reference)
import jax
import jax.numpy as jnp
from jax.experimental import pallas as pl
from jax.experimental.pallas import tpu as pltpu


def _silu(v):
    return v * jax.nn.sigmoid(v)


def _moe_body(x_ref, wg_ref, ws1_ref, ws2_ref, w1_ref, w2_ref, out_ref, acc_ref):
    e = pl.program_id(1)
    E = pl.num_programs(1)
    xb = x_ref[...]

    logits = jax.lax.dot_general(
        xb, wg_ref[...], (((1,), (1,)), ((), ())),
        preferred_element_type=jnp.float32)
    s = jax.nn.sigmoid(logits)
    m = jnp.max(s, axis=1)
    ii = jax.lax.broadcasted_iota(jnp.int32, s.shape, 1)
    eid = jnp.min(jnp.where(s >= m[:, None], ii, E), axis=1)
    w = m / (m + 1e-20)
    c = jnp.where(eid == e, w, 0.0)

    h = _silu(jax.lax.dot_general(
        xb, w1_ref[0], (((1,), (1,)), ((), ())),
        preferred_element_type=jnp.float32))
    y = jax.lax.dot_general(
        h, w2_ref[0], (((1,), (1,)), ((), ())),
        preferred_element_type=jnp.float32)
    contrib = c[:, None] * y

    @pl.when(e == 0)
    def _init():
        sh = _silu(jax.lax.dot_general(
            xb, ws1_ref[...], (((1,), (1,)), ((), ())),
            preferred_element_type=jnp.float32))
        shared = jax.lax.dot_general(
            sh, ws2_ref[...], (((1,), (1,)), ((), ())),
            preferred_element_type=jnp.float32)
        acc_ref[...] = shared + contrib

    @pl.when(e > 0)
    def _accum():
        acc_ref[...] += contrib

    @pl.when(e == E - 1)
    def _emit():
        out_ref[...] = acc_ref[...]


def kernel(x, Wg, Ws1, Ws2, W1, W2):
    B, T, D = x.shape
    E, F, _ = W1.shape
    FS = Ws1.shape[0]
    xf = x.reshape(T, D)

    TB = 256
    nt = T // TB

    out = pl.pallas_call(
        _moe_body,
        grid=(nt, E),
        in_specs=[
            pl.BlockSpec((TB, D), lambda t, e: (t, 0)),
            pl.BlockSpec((E, D), lambda t, e: (0, 0)),
            pl.BlockSpec((FS, D), lambda t, e: (0, 0)),
            pl.BlockSpec((D, FS), lambda t, e: (0, 0)),
            pl.BlockSpec((1, F, D), lambda t, e: (e, 0, 0)),
            pl.BlockSpec((1, D, F), lambda t, e: (e, 0, 0)),
        ],
        out_specs=pl.BlockSpec((TB, D), lambda t, e: (t, 0)),
        out_shape=jax.ShapeDtypeStruct((T, D), jnp.float32),
        scratch_shapes=[pltpu.VMEM((TB, D), jnp.float32)],
        compiler_params=pltpu.CompilerParams(
            dimension_semantics=("arbitrary", "arbitrary")),
    )(xf, Wg, Ws1, Ws2, W1, W2)
    return out.reshape(B, T, D)


# speedup vs baseline: 1.4976x; 1.4976x over previous
import functools

import jax
import jax.numpy as jnp
from jax import lax
from jax.experimental import pallas as pl
from jax.experimental.pallas import tpu as pltpu
from jax.experimental.pallas import tpu_sc as plsc

_NC = 2
_NS = 16
_NW = _NC * _NS


def _silu(v):
    return v * jax.nn.sigmoid(v)


def _dot_t(a, b):
    return jax.lax.dot_general(a, b, (((1,), (1,)), ((), ())),
                               preferred_element_type=jnp.float32)


def _router_body(x_ref, wg_ref, eid_ref, w_ref):
    E = wg_ref.shape[0]
    s = jax.nn.sigmoid(_dot_t(x_ref[...], wg_ref[...]))
    m = jnp.max(s, axis=1)
    ii = jax.lax.broadcasted_iota(jnp.int32, s.shape, 1)
    eid = jnp.min(jnp.where(s >= m[:, None], ii, E), axis=1)
    eid_ref[...] = eid
    w_ref[...] = m / (m + 1e-20)


def _router(xf, Wg):
    T, D = xf.shape
    E = Wg.shape[0]
    return pl.pallas_call(
        _router_body,
        out_shape=(jax.ShapeDtypeStruct((T,), jnp.int32),
                   jax.ShapeDtypeStruct((T,), jnp.float32)),
    )(xf, Wg)


def _make_sc_gather(n_rows, n_tab, d, chunk):
    assert n_rows % (_NW * chunk) == 0
    n_chunks = n_rows // (_NW * chunk)
    mesh = plsc.VectorSubcoreMesh(core_axis_name="c", subcore_axis_name="s")

    @functools.partial(
        pl.kernel, mesh=mesh,
        out_type=jax.ShapeDtypeStruct((n_rows, d), jnp.float32),
        scratch_types=[
            pltpu.VMEM((chunk,), jnp.int32),
            pltpu.VMEM((chunk, d), jnp.float32),
            pltpu.SemaphoreType.DMA,
        ],
    )
    def k(idx_hbm, tab_hbm, out_hbm, idx_v, rows_v, sem):
        wid = lax.axis_index("s") * _NC + lax.axis_index("c")
        base0 = wid * (n_chunks * chunk)
        for c in range(n_chunks):
            base = base0 + c * chunk
            pltpu.sync_copy(idx_hbm.at[pl.ds(base, chunk)], idx_v)
            pltpu.async_copy(tab_hbm.at[idx_v], rows_v, sem).wait()
            pltpu.sync_copy(rows_v, out_hbm.at[pl.ds(base, chunk)])

    return k


def _group_body(be_ref, nba_ref, x_ref, wsl_ref, ws1_ref, ws2_ref,
                w1_ref, w2_ref, out_ref):
    g = pl.program_id(0)

    @pl.when(g < nba_ref[0])
    def _():
        xb = x_ref[...]
        h = _silu(_dot_t(xb, w1_ref[0]))
        y = _dot_t(h, w2_ref[0])
        sh = _silu(_dot_t(xb, ws1_ref[...]))
        shared = _dot_t(sh, ws2_ref[...])
        out_ref[...] = shared + wsl_ref[0, 0, :][:, None] * y


def _grouped_mlp(x_pad, w_slot3, Ws1, Ws2, W1, W2, block_expert, nb_act, blk):
    nslot, D = x_pad.shape
    E, F, _ = W1.shape
    FS = Ws1.shape[0]
    nb = nslot // blk
    grid_spec = pltpu.PrefetchScalarGridSpec(
        num_scalar_prefetch=2,
        grid=(nb,),
        in_specs=[
            pl.BlockSpec((blk, D), lambda g, be, nba: (g, 0)),
            pl.BlockSpec((1, 1, blk), lambda g, be, nba: (g, 0, 0)),
            pl.BlockSpec((FS, D), lambda g, be, nba: (0, 0)),
            pl.BlockSpec((D, FS), lambda g, be, nba: (0, 0)),
            pl.BlockSpec((1, F, D), lambda g, be, nba: (be[g], 0, 0)),
            pl.BlockSpec((1, D, F), lambda g, be, nba: (be[g], 0, 0)),
        ],
        out_specs=pl.BlockSpec((blk, D), lambda g, be, nba: (g, 0)),
    )
    return pl.pallas_call(
        _group_body,
        grid_spec=grid_spec,
        out_shape=jax.ShapeDtypeStruct((nslot, D), jnp.float32),
        compiler_params=pltpu.CompilerParams(
            dimension_semantics=("arbitrary",)),
    )(block_expert, nb_act, x_pad, w_slot3, Ws1, Ws2, W1, W2)


_BLK = 128


def kernel(x, Wg, Ws1, Ws2, W1, W2):
    B, T, D = x.shape
    E, F, _ = W1.shape
    xf = x.reshape(T, D)
    blk = _BLK
    nb = T // blk + E
    nslot = nb * blk

    eid, w = _router(xf, Wg)

    perm = jnp.argsort(eid)
    counts = jnp.bincount(eid, length=E)
    offs = jnp.concatenate([jnp.zeros((1,), jnp.int32),
                            jnp.cumsum(counts)[:-1].astype(jnp.int32)])
    nblocks_e = (counts + blk - 1) // blk
    pad_offs = jnp.concatenate([jnp.zeros((1,), jnp.int32),
                                jnp.cumsum(nblocks_e * blk)[:-1].astype(jnp.int32)])
    nb_act = jnp.sum(nblocks_e).astype(jnp.int32)
    block_expert = jnp.repeat(jnp.arange(E, dtype=jnp.int32), nblocks_e,
                              total_repeat_length=nb)
    last_e = block_expert[jnp.maximum(nb_act - 1, 0)]
    block_expert = jnp.where(jnp.arange(nb) < nb_act, block_expert, last_e)

    eperm = eid[perm]
    p = jnp.arange(T, dtype=jnp.int32)
    q = pad_offs[eperm] + p - offs[eperm]
    tok_of_slot = jnp.zeros((nslot,), jnp.int32).at[q].set(perm.astype(jnp.int32))
    w_slot = jnp.zeros((nslot,), jnp.float32).at[q].set(w[perm])
    slot_of_tok = jnp.zeros((T,), jnp.int32).at[perm].set(q)

    x_pad = _make_sc_gather(nslot, T, D, 64)(tok_of_slot, xf)

    out_pad = _grouped_mlp(x_pad, w_slot.reshape(nb, 1, blk), Ws1, Ws2,
                           W1, W2, block_expert, nb_act[None], blk)

    out = _make_sc_gather(T, nslot, D, 64)(slot_of_tok, out_pad)
    return out.reshape(B, T, D)


# speedup vs baseline: 2.1290x; 1.4216x over previous
import functools

import jax
import jax.numpy as jnp
from jax import lax
from jax.experimental import pallas as pl
from jax.experimental.pallas import tpu as pltpu
from jax.experimental.pallas import tpu_sc as plsc

_NC = 2
_NS = 16
_NW = _NC * _NS


def _silu(v):
    return v * jax.nn.sigmoid(v)


def _dot_t(a, b):
    return jax.lax.dot_general(a, b, (((1,), (1,)), ((), ())),
                               preferred_element_type=jnp.float32)


def _router_body(x_ref, wg_ref, eid_ref, w_ref):
    E = wg_ref.shape[0]
    s = jax.nn.sigmoid(_dot_t(x_ref[...], wg_ref[...]))
    m = jnp.max(s, axis=1)
    ii = jax.lax.broadcasted_iota(jnp.int32, s.shape, 1)
    eid = jnp.min(jnp.where(s >= m[:, None], ii, E), axis=1)
    eid_ref[...] = eid
    w_ref[...] = m / (m + 1e-20)


def _router(xf, Wg):
    T, D = xf.shape
    E = Wg.shape[0]
    return pl.pallas_call(
        _router_body,
        out_shape=(jax.ShapeDtypeStruct((T,), jnp.int32),
                   jax.ShapeDtypeStruct((T,), jnp.float32)),
    )(xf, Wg)


def _make_sc_gather(n_rows, n_tab, d, chunk):
    assert n_rows % (_NW * chunk) == 0
    n_chunks = n_rows // (_NW * chunk)
    mesh = plsc.VectorSubcoreMesh(core_axis_name="c", subcore_axis_name="s")

    @functools.partial(
        pl.kernel, mesh=mesh,
        out_type=jax.ShapeDtypeStruct((n_rows, d), jnp.float32),
        scratch_types=[
            pltpu.VMEM((chunk,), jnp.int32),
            pltpu.VMEM((chunk, d), jnp.float32),
            pltpu.SemaphoreType.DMA,
        ],
    )
    def k(idx_hbm, tab_hbm, out_hbm, idx_v, rows_v, sem):
        wid = lax.axis_index("s") * _NC + lax.axis_index("c")
        base0 = wid * (n_chunks * chunk)
        for c in range(n_chunks):
            base = base0 + c * chunk
            pltpu.sync_copy(idx_hbm.at[pl.ds(base, chunk)], idx_v)
            pltpu.async_copy(tab_hbm.at[idx_v], rows_v, sem).wait()
            pltpu.sync_copy(rows_v, out_hbm.at[pl.ds(base, chunk)])

    return k


def _group_body(be_ref, nba_ref, x_ref, wsl_ref, ws1_ref, ws2_ref,
                w1_ref, w2_ref, out_ref):
    g = pl.program_id(0)

    @pl.when(g < nba_ref[0])
    def _():
        xb = x_ref[...]
        h = _silu(_dot_t(xb, w1_ref[0]))
        y = _dot_t(h, w2_ref[0])
        sh = _silu(_dot_t(xb, ws1_ref[...]))
        shared = _dot_t(sh, ws2_ref[...])
        out_ref[...] = shared + wsl_ref[0, 0, :][:, None] * y


def _grouped_mlp(x_pad, w_slot3, Ws1, Ws2, W1, W2, block_expert, nb_act, blk):
    nslot, D = x_pad.shape
    E, F, _ = W1.shape
    FS = Ws1.shape[0]
    nb = nslot // blk
    grid_spec = pltpu.PrefetchScalarGridSpec(
        num_scalar_prefetch=2,
        grid=(nb,),
        in_specs=[
            pl.BlockSpec((blk, D), lambda g, be, nba: (g, 0)),
            pl.BlockSpec((1, 1, blk), lambda g, be, nba: (g, 0, 0)),
            pl.BlockSpec((FS, D), lambda g, be, nba: (0, 0)),
            pl.BlockSpec((D, FS), lambda g, be, nba: (0, 0)),
            pl.BlockSpec((1, F, D), lambda g, be, nba: (be[g], 0, 0)),
            pl.BlockSpec((1, D, F), lambda g, be, nba: (be[g], 0, 0)),
        ],
        out_specs=pl.BlockSpec((blk, D), lambda g, be, nba: (g, 0)),
    )
    return pl.pallas_call(
        _group_body,
        grid_spec=grid_spec,
        out_shape=jax.ShapeDtypeStruct((nslot, D), jnp.float32),
        compiler_params=pltpu.CompilerParams(
            dimension_semantics=("arbitrary",)),
    )(block_expert, nb_act, x_pad, w_slot3, Ws1, Ws2, W1, W2)


_BLK = 128


def kernel(x, Wg, Ws1, Ws2, W1, W2):
    B, T, D = x.shape
    E, F, _ = W1.shape
    xf = x.reshape(T, D)
    blk = _BLK
    nb = T // blk + E
    nslot = nb * blk

    eid, w = _router(xf, Wg)

    perm = jnp.argsort(eid)
    counts = jnp.bincount(eid, length=E)
    offs = jnp.concatenate([jnp.zeros((1,), jnp.int32),
                            jnp.cumsum(counts)[:-1].astype(jnp.int32)])
    nblocks_e = (counts + blk - 1) // blk
    pad_offs = jnp.concatenate([jnp.zeros((1,), jnp.int32),
                                jnp.cumsum(nblocks_e * blk)[:-1].astype(jnp.int32)])
    nb_act = jnp.sum(nblocks_e).astype(jnp.int32)
    block_expert = jnp.repeat(jnp.arange(E, dtype=jnp.int32), nblocks_e,
                              total_repeat_length=nb)
    last_e = block_expert[jnp.maximum(nb_act - 1, 0)]
    block_expert = jnp.where(jnp.arange(nb) < nb_act, block_expert, last_e)

    eperm = eid[perm]
    p = jnp.arange(T, dtype=jnp.int32)
    q = pad_offs[eperm] + p - offs[eperm]
    pad_fill = jnp.arange(nslot, dtype=jnp.int32) % T
    tok_of_slot = pad_fill.at[q].set(perm.astype(jnp.int32))
    w_slot = jnp.zeros((nslot,), jnp.float32).at[q].set(w[perm])
    slot_of_tok = jnp.zeros((T,), jnp.int32).at[perm].set(q)

    x_pad = _make_sc_gather(nslot, T, D, 64)(tok_of_slot, xf)

    out_pad = _grouped_mlp(x_pad, w_slot.reshape(nb, 1, blk), Ws1, Ws2,
                           W1, W2, block_expert, nb_act[None], blk)

    out = _make_sc_gather(T, nslot, D, 64)(slot_of_tok, out_pad)
    return out.reshape(B, T, D)
